# baseline (speedup 1.0000x reference)
import functools
import jax
import jax.numpy as jnp
from jax import lax
from jax.experimental import pallas as pl
from jax.experimental.pallas import tpu as pltpu, tpu_sc as plsc

_B = 4
_S = 8192
_D = 1024
_L = 16
_CH = 16


def _make_sc_add():
    info = plsc.get_sparse_core_info()
    NC, NS = info.num_cores, info.num_subcores
    NW = NC * NS
    pos_per_w = _S // NW
    n_chunks = pos_per_w // _CH
    mesh = plsc.VectorSubcoreMesh(core_axis_name="c", subcore_axis_name="s")

    @functools.partial(
        pl.kernel,
        mesh=mesh,
        out_type=jax.ShapeDtypeStruct((_B, _S, _D), jnp.float32),
        scratch_types=[
            pltpu.VMEM((_CH, _D), jnp.float32),
            pltpu.VMEM((_CH, _D), jnp.float32),
        ],
    )
    def sc_add(x_hbm, t_hbm, out_hbm, tbuf, xbuf):
        wid = lax.axis_index("s") * NC + lax.axis_index("c")
        base = wid * pos_per_w

        def t_loop(t, carry):
            s0 = base + t * _CH
            pltpu.sync_copy(t_hbm.at[pl.ds(s0, _CH), :], tbuf)

            def b_loop(b, carry2):
                pltpu.sync_copy(x_hbm.at[b, pl.ds(s0, _CH), :], xbuf)

                def i_loop(i, carry3):
                    for j in range(_D // _L):
                        sl = pl.ds(j * _L, _L)
                        xbuf[i, sl] = xbuf[i, sl] + tbuf[i, sl]
                    return carry3

                lax.fori_loop(0, _CH, i_loop, 0)
                pltpu.sync_copy(xbuf, out_hbm.at[b, pl.ds(s0, _CH), :])
                return carry2

            lax.fori_loop(0, _B, b_loop, 0)
            return carry

        lax.fori_loop(0, n_chunks, t_loop, 0)

    return sc_add


_sc_add = _make_sc_add()


def kernel(x, table):
    return _sc_add(x, table)


# speedup vs baseline: 1.7562x; 1.7562x over previous
import functools
import jax
import jax.numpy as jnp
from jax import lax
from jax.experimental import pallas as pl
from jax.experimental.pallas import tpu as pltpu, tpu_sc as plsc

_B = 4
_S = 8192
_D = 1024
_L = 16
_CH = 16


def _make_sc_add():
    info = plsc.get_sparse_core_info()
    NC, NS = info.num_cores, info.num_subcores
    NW = NC * NS
    pos_per_w = _S // NW
    n_steps = (pos_per_w // _CH) * _B
    mesh = plsc.VectorSubcoreMesh(core_axis_name="c", subcore_axis_name="s")

    buf = lambda: pltpu.VMEM((_CH, _D), jnp.float32)

    @functools.partial(
        pl.kernel,
        mesh=mesh,
        out_type=jax.ShapeDtypeStruct((_B, _S, _D), jnp.float32),
        scratch_types=[
            buf(), buf(),
            buf(), buf(),
            buf(), buf(),
            pltpu.SemaphoreType.DMA, pltpu.SemaphoreType.DMA,
            pltpu.SemaphoreType.DMA, pltpu.SemaphoreType.DMA,
            pltpu.SemaphoreType.DMA, pltpu.SemaphoreType.DMA,
        ],
    )
    def sc_add(x_hbm, t_hbm, out_hbm, xb0, xb1, tb0, tb1, ob0, ob1,
               xsem0, xsem1, tsem0, tsem1, osem0, osem1):
        wid = lax.axis_index("s") * NC + lax.axis_index("c")
        base = wid * pos_per_w

        def start_loads(k, xb, tb, xsem, tsem):
            t = k // _B
            b = k % _B
            s0 = base + t * _CH
            pltpu.make_async_copy(x_hbm.at[b, pl.ds(s0, _CH), :], xb, xsem).start()
            pltpu.make_async_copy(t_hbm.at[pl.ds(s0, _CH), :], tb, tsem).start()

        def wait_loads(xb, tb, xsem, tsem):
            pltpu.make_async_copy(x_hbm.at[0, pl.ds(base, _CH), :], xb, xsem).wait()
            pltpu.make_async_copy(t_hbm.at[pl.ds(base, _CH), :], tb, tsem).wait()

        def compute(xb, tb, ob):
            def i_loop(i, carry):
                for j in range(_D // _L):
                    sl = pl.ds(j * _L, _L)
                    ob[i, sl] = xb[i, sl] + tb[i, sl]
                return carry

            lax.fori_loop(0, _CH, i_loop, 0)

        def start_store(k, ob, osem):
            t = k // _B
            b = k % _B
            s0 = base + t * _CH
            pltpu.make_async_copy(ob, out_hbm.at[b, pl.ds(s0, _CH), :], osem).start()

        def wait_store(ob, osem):
            pltpu.make_async_copy(ob, out_hbm.at[0, pl.ds(base, _CH), :], osem).wait()

        start_loads(0, xb0, tb0, xsem0, tsem0)
        start_loads(1, xb1, tb1, xsem1, tsem1)

        def half_step(m, k, xb, tb, ob, xsem, tsem, osem):
            wait_loads(xb, tb, xsem, tsem)

            @pl.when(m > 0)
            def _():
                wait_store(ob, osem)

            compute(xb, tb, ob)
            start_store(k, ob, osem)

            @pl.when(k + 2 < n_steps)
            def _():
                start_loads(k + 2, xb, tb, xsem, tsem)

        def m_loop(m, carry):
            half_step(m, 2 * m, xb0, tb0, ob0, xsem0, tsem0, osem0)
            half_step(m, 2 * m + 1, xb1, tb1, ob1, xsem1, tsem1, osem1)
            return carry

        lax.fori_loop(0, n_steps // 2, m_loop, 0)
        wait_store(ob0, osem0)
        wait_store(ob1, osem1)

    return sc_add


_sc_add = _make_sc_add()


def kernel(x, table):
    return _sc_add(x, table)


# speedup vs baseline: 3.0529x; 1.7383x over previous
import jax
import jax.numpy as jnp
from jax.experimental import pallas as pl

_BS = 1024


def _add_block(x_ref, t_ref, o_ref):
    i = pl.program_id(0)
    o_ref[0] = x_ref[0] + t_ref[pl.ds(i * _BS, _BS), :]


def kernel(x, table):
    B, S, D = x.shape
    grid = (S // _BS, B)
    return pl.pallas_call(
        _add_block,
        grid=grid,
        in_specs=[
            pl.BlockSpec((1, _BS, D), lambda i, b: (b, i, 0)),
            pl.BlockSpec((S, D), lambda i, b: (0, 0)),
        ],
        out_specs=pl.BlockSpec((1, _BS, D), lambda i, b: (b, i, 0)),
        out_shape=jax.ShapeDtypeStruct(x.shape, x.dtype),
    )(x, table)


# speedup vs baseline: 3.1167x; 1.0209x over previous
import jax
import jax.numpy as jnp
from jax.experimental import pallas as pl

_BS = 2048


def _add_block(x_ref, t_ref, o_ref):
    o_ref[...] = x_ref[...] + t_ref[...]


def kernel(x, table):
    B, S, D = x.shape
    grid = (S // _BS, B)
    return pl.pallas_call(
        _add_block,
        grid=grid,
        in_specs=[
            pl.BlockSpec((1, _BS, D), lambda i, b: (b, i, 0)),
            pl.BlockSpec((_BS, D), lambda i, b: (i, 0)),
        ],
        out_specs=pl.BlockSpec((1, _BS, D), lambda i, b: (b, i, 0)),
        out_shape=jax.ShapeDtypeStruct(x.shape, x.dtype),
    )(x, table)
